# baseline (speedup 1.0000x reference)
import functools

import jax
import jax.numpy as jnp
from jax import lax
from jax.experimental import pallas as pl
from jax.experimental.pallas import tpu as pltpu
from jax.experimental.pallas import tpu_sc as plsc

_K = 10
_KP = 16
_D = 128


def _proj_t_kernel(table_ref, w_ref, b_ref, mt_ref):
    h = jnp.maximum(table_ref[...], 0.0)
    mt = lax.dot_general(w_ref[...], h, (((1,), (1,)), ((), ())),
                         preferred_element_type=jnp.float32)
    mt_ref[...] = mt + b_ref[...]


@functools.lru_cache(maxsize=None)
def _make_sc_lookup(B: int, L: int):
    info = plsc.get_sparse_core_info()
    num_cores = info.num_cores
    num_workers = info.num_cores * info.num_subcores
    n_btiles = B // 128
    bt_per_w = n_btiles // num_workers
    n_ltiles = L // 8

    mesh = plsc.VectorSubcoreMesh(core_axis_name="c", subcore_axis_name="s")

    nbuf = 4
    n_groups = bt_per_w * n_ltiles

    @functools.partial(
        pl.kernel,
        mesh=mesh,
        out_type=jax.ShapeDtypeStruct((_K, L, B), jnp.float32),
        scratch_types=[
            pltpu.VMEM((_K, _KP), jnp.float32),
            pltpu.VMEM((128, L), jnp.int32),
            pltpu.VMEM((nbuf, _K, 8, 128), jnp.float32),
            [pltpu.SemaphoreType.DMA] * nbuf,
        ],
        compiler_params=pltpu.CompilerParams(needs_layout_passes=False),
    )
    def sc_lookup(mt_hbm, x_hbm, out_hbm, mt_v, xband_v, otile_v, sems):
        wid = lax.axis_index("s") * num_cores + lax.axis_index("c")
        bt0 = wid * bt_per_w

        pltpu.sync_copy(mt_hbm, mt_v)

        iota = lax.iota(jnp.int32, 16)
        ridx = [iota + (c * 16) for c in range(8)]
        kvec = [jnp.zeros((16,), jnp.int32) + k for k in range(_K)]
        zero16 = jnp.zeros((16,), jnp.int32)

        def compute_group(lt, buf):
            def lv_body(lv, carry):
                cidx = zero16 + (lt * 8 + lv)
                for c in range(8):
                    xg = plsc.load_gather(xband_v, [ridx[c], cidx])
                    for k in range(_K):
                        val = plsc.load_gather(mt_v, [kvec[k], xg])
                        otile_v[buf, k, lv, pl.ds(c * 16, 16)] = val
                return carry
            lax.fori_loop(0, 8, lv_body, 0)

        def fire(bt, lt, buf):
            for k in range(_K):
                pltpu.async_copy(
                    otile_v.at[buf, k],
                    out_hbm.at[k, pl.ds(lt * 8, 8), pl.ds(bt * 128, 128)],
                    sems[buf])

        def drain(buf):
            pltpu.make_async_copy(
                out_hbm.at[:, pl.ds(0, 8), pl.ds(0, 128)],
                otile_v.at[buf],
                sems[buf]).wait()

        def stage_band(bt):
            pltpu.sync_copy(x_hbm.at[pl.ds(bt * 128, 128), :], xband_v)

        stage_band(bt0)
        for j in range(nbuf):
            fire(bt0, j, j)

        def pair_body(i, carry):
            for j in range(nbuf):
                gg = i * nbuf + j
                band = lax.shift_right_logical(gg * 41, 10)
                lt = gg - band * 25
                bt = bt0 + band

                @pl.when(jnp.logical_and(lt == 0, gg > 0))
                def _():
                    stage_band(bt)

                drain(j)
                compute_group(lt, j)
                fire(bt, lt, j)
            return carry

        lax.fori_loop(0, n_groups // nbuf, pair_body, 0)

        for j in range(nbuf):
            drain(j)

    return sc_lookup


def kernel(x, table, W, b):
    B, L = x.shape

    table_p = jnp.zeros((_KP, _D), jnp.float32).at[:_K].set(table)
    mt = pl.pallas_call(
        _proj_t_kernel,
        out_shape=jax.ShapeDtypeStruct((_K, _KP), jnp.float32),
    )(table_p, W, b.reshape(_K, 1))

    out3 = _make_sc_lookup(B, L)(mt, x.astype(jnp.int32))
    return jnp.transpose(out3, (2, 1, 0))


# speedup vs baseline: 1.4285x; 1.4285x over previous
import functools

import jax
import jax.numpy as jnp
from jax import lax
from jax.experimental import pallas as pl
from jax.experimental.pallas import tpu as pltpu
from jax.experimental.pallas import tpu_sc as plsc

_K = 10
_KP = 16
_D = 128


def _proj_t_kernel(table_ref, w_ref, b_ref, mt_ref):
    h = jnp.maximum(table_ref[...], 0.0)
    mt = lax.dot_general(w_ref[...], h, (((1,), (1,)), ((), ())),
                         preferred_element_type=jnp.float32)
    mt_ref[...] = mt + b_ref[...]


@functools.lru_cache(maxsize=None)
def _make_sc_lookup(B: int, L: int):
    info = plsc.get_sparse_core_info()
    num_cores = info.num_cores
    num_workers = info.num_cores * info.num_subcores
    n_btiles = B // 128
    bt_per_w = n_btiles // num_workers
    n_ltiles = L // 8

    mesh = plsc.VectorSubcoreMesh(core_axis_name="c", subcore_axis_name="s")

    nbuf = 4
    n_groups = bt_per_w * n_ltiles

    @functools.partial(
        pl.kernel,
        mesh=mesh,
        out_type=jax.ShapeDtypeStruct((_K, L, B), jnp.float32),
        scratch_types=[
            pltpu.VMEM((_K, _KP), jnp.float32),
            pltpu.VMEM((128, L), jnp.int32),
            pltpu.VMEM((nbuf, _K, 8, 128), jnp.float32),
            [pltpu.SemaphoreType.DMA] * nbuf,
        ],
        compiler_params=pltpu.CompilerParams(needs_layout_passes=False),
    )
    def sc_lookup(mt_hbm, x_hbm, out_hbm, mt_v, xband_v, otile_v, sems):
        wid = lax.axis_index("s") * num_cores + lax.axis_index("c")
        bt0 = wid * bt_per_w

        pltpu.sync_copy(mt_hbm, mt_v)

        iota = lax.iota(jnp.int32, 16)
        ridx = [iota + (c * 16) for c in range(8)]
        kvec = [jnp.zeros((16,), jnp.int32) + k for k in range(_K)]
        zero16 = jnp.zeros((16,), jnp.int32)

        def compute_group(lt, buf):
            @plsc.parallel_loop(0, 8, unroll=4)
            def lv_body(lv):
                cidx = zero16 + (lt * 8 + lv)
                for c in range(8):
                    xg = plsc.load_gather(xband_v, [ridx[c], cidx])
                    for k in range(_K):
                        val = plsc.load_gather(mt_v, [kvec[k], xg])
                        otile_v[buf, k, lv, pl.ds(c * 16, 16)] = val

        def fire(bt, lt, buf):
            for k in range(_K):
                pltpu.async_copy(
                    otile_v.at[buf, k],
                    out_hbm.at[k, pl.ds(lt * 8, 8), pl.ds(bt * 128, 128)],
                    sems[buf])

        def drain(buf):
            pltpu.make_async_copy(
                out_hbm.at[:, pl.ds(0, 8), pl.ds(0, 128)],
                otile_v.at[buf],
                sems[buf]).wait()

        def stage_band(bt):
            pltpu.sync_copy(x_hbm.at[pl.ds(bt * 128, 128), :], xband_v)

        stage_band(bt0)
        for j in range(nbuf):
            fire(bt0, j, j)

        def pair_body(i, carry):
            for j in range(nbuf):
                gg = i * nbuf + j
                band = lax.shift_right_logical(gg * 41, 10)
                lt = gg - band * 25
                bt = bt0 + band

                @pl.when(jnp.logical_and(lt == 0, gg > 0))
                def _():
                    stage_band(bt)

                drain(j)
                compute_group(lt, j)
                fire(bt, lt, j)
            return carry

        lax.fori_loop(0, n_groups // nbuf, pair_body, 0)

        for j in range(nbuf):
            drain(j)

    return sc_lookup


def kernel(x, table, W, b):
    B, L = x.shape

    table_p = jnp.zeros((_KP, _D), jnp.float32).at[:_K].set(table)
    mt = pl.pallas_call(
        _proj_t_kernel,
        out_shape=jax.ShapeDtypeStruct((_K, _KP), jnp.float32),
    )(table_p, W, b.reshape(_K, 1))

    out3 = _make_sc_lookup(B, L)(mt, x.astype(jnp.int32))
    return jnp.transpose(out3, (2, 1, 0))
